# baseline (speedup 1.0000x reference)
import functools

import jax
import jax.numpy as jnp
from jax import lax
from jax.experimental import pallas as pl
from jax.experimental.pallas import tpu as pltpu
from jax.experimental.pallas import tpu_sc as plsc

B = 32
L_SEQ = 512
N_VAR = 321
N_MARK = 4
N_TOK = N_VAR + N_MARK
N_PAD = 328
D_MODEL = 512
D_FF = 2048
N_HEADS = 8
D_HEAD = 64
E_LAYERS = 2
PRED_LEN = 336
TOPK = 10
FLAT = N_PAD * PRED_LEN
CHUNK = 128 * 41
N_CHUNKS = FLAT // CHUNK

GB = 2

_BF = jnp.bfloat16
_F32 = jnp.float32


def _dot(a, b, precision=None):
    return lax.dot_general(a, b, (((1,), (0,)), ((), ())),
                           preferred_element_type=_F32, precision=precision)


def _dot_nt(a, b):
    return lax.dot_general(a, b, (((1,), (1,)), ((), ())),
                           preferred_element_type=_F32)


def _ln(x, g, b, eps=1e-5):
    m = jnp.mean(x, axis=1, keepdims=True)
    v = jnp.mean(x * x, axis=1, keepdims=True) - m * m
    return (x - m) / jnp.sqrt(v + eps) * g + b


def _enc_kernel(xt_ref, Wemb_ref, bemb_ref, Wqkv_ref, bqkv_ref, Wo_ref,
                bo_ref, Wc1_ref, bc1_ref, Wc2_ref, bc2_ref, g1_ref, be1_ref,
                g2_ref, be2_ref, gF_ref, bF_ref, Wp_ref, bp_ref, out_ref):
    x = xt_ref[...].reshape(GB * N_PAD, D_MODEL)
    h = _dot(x, Wemb_ref[...]) + bemb_ref[...]

    col = lax.broadcasted_iota(jnp.int32, (1, N_PAD), 1)
    key_mask = jnp.where(col < N_TOK, 0.0, -1e30).astype(_F32)

    for l in range(E_LAYERS):
        hb = h.astype(_BF)
        qkv = _dot(hb, Wqkv_ref[l]) + bqkv_ref[l]
        qkvb = qkv.astype(_BF)
        qb = qkvb[:, 0:D_MODEL]
        kb = qkvb[:, D_MODEL:2 * D_MODEL]
        vb = qkvb[:, 2 * D_MODEL:3 * D_MODEL]
        rows = []
        for eb in range(GB):
            rs = slice(eb * N_PAD, (eb + 1) * N_PAD)
            heads = []
            for hd in range(N_HEADS):
                sl = slice(hd * D_HEAD, (hd + 1) * D_HEAD)
                s = _dot_nt(qb[rs, sl], kb[rs, sl]) * 0.125
                e = jnp.exp(s + key_mask)
                a = e / jnp.sum(e, axis=1, keepdims=True)
                heads.append(_dot(a.astype(_BF), vb[rs, sl]))
            rows.append(jnp.concatenate(heads, axis=1))
        o = jnp.concatenate(rows, axis=0)
        o = _dot(o.astype(_BF), Wo_ref[l]) + bo_ref[l]
        h = _ln(h + o, g1_ref[l], be1_ref[l])
        y = _dot(h.astype(_BF), Wc1_ref[l]) + bc1_ref[l]
        y = jax.nn.gelu(y)
        y = _dot(y.astype(_BF), Wc2_ref[l]) + bc2_ref[l]
        h = _ln(h + y, g2_ref[l], be2_ref[l])

    h = _ln(h, gF_ref[...], bF_ref[...])
    pt = _dot(h.astype(_BF), Wp_ref[...]) + bp_ref[...]
    colv = lax.broadcasted_iota(jnp.int32, (1, N_PAD), 1)
    for eb in range(GB):
        ptT = jnp.swapaxes(pt[eb * N_PAD:(eb + 1) * N_PAD], 0, 1)
        out_ref[eb] = jnp.where(colv < N_VAR, ptT, 0.0)


def _sim_kernel(p_ref, sim_ref, qnb_ref):
    ss = jnp.zeros((B, 1), _F32)
    for i in range(N_CHUNKS):
        x = p_ref[:, pl.ds(i * CHUNK, CHUNK)]
        ss = ss + jnp.sum(x * x, axis=1, keepdims=True)
    denom = jnp.sqrt(ss) + 1e-8

    for i in range(N_CHUNKS):
        x = p_ref[:, pl.ds(i * CHUNK, CHUNK)]
        qnb_ref[:, pl.ds(i * CHUNK, CHUNK)] = (x / denom).astype(_BF)

    sim = jnp.zeros((B, B), _F32)
    for i in range(N_CHUNKS):
        c = qnb_ref[:, pl.ds(i * CHUNK, CHUNK)]
        sim = sim + _dot_nt(c, c)
    sim_ref[...] = sim


def _select_sc(sim_hbm, a_hbm, row_v, out_v):
    wid = lax.axis_index("s") * 2 + lax.axis_index("c")
    base = wid * B
    pltpu.sync_copy(sim_hbm.at[pl.ds(base, B)], row_v.at[pl.ds(0, B)])
    pltpu.sync_copy(sim_hbm.at[pl.ds(base, B)], row_v.at[pl.ds(B, B)])
    lane = lax.iota(jnp.int32, 16)
    a1 = row_v[pl.ds(0, 16)]
    a2 = row_v[pl.ds(16, 16)]
    r1 = jnp.zeros((16,), jnp.int32)
    r2 = jnp.zeros((16,), jnp.int32)
    one = jnp.ones((16,), jnp.int32)
    zero = jnp.zeros((16,), jnp.int32)
    for sft in range(1, B):
        w1 = row_v[pl.ds(sft, 16)]
        w2 = row_v[pl.ds(16 + sft, 16)]
        beat1 = (w1 > a1) | ((w1 == a1) & (lane >= B - sft))
        beat2 = (w2 > a2) | ((w2 == a2) & (lane >= 16 - sft))
        r1 = r1 + jnp.where(beat1, one, zero)
        r2 = r2 + jnp.where(beat2, one, zero)
    tenth = jnp.full((16,), 1.0 / TOPK, _F32)
    zf = jnp.zeros((16,), _F32)
    out_v[pl.ds(0, 16)] = jnp.where(r1 < TOPK, tenth, zf)
    out_v[pl.ds(16, 16)] = jnp.where(r2 < TOPK, tenth, zf)
    pltpu.sync_copy(out_v, a_hbm.at[pl.ds(base, B)])


def _fuse_kernel(p_ref, a_ref, out_ref):
    A = a_ref[...]
    for i in range(N_CHUNKS):
        x = p_ref[:, pl.ds(i * CHUNK, CHUNK)]
        f = _dot(A, x, precision=lax.Precision.HIGHEST)
        out_ref[:, pl.ds(i * CHUNK, CHUNK)] = 0.5 * x + 0.5 * f


@functools.partial(jax.jit, static_argnames=("interpret",))
def _run(args, interpret=False):
    (x_enc, x_mark_enc, W_emb, b_emb, Wq, bq, Wk, bk, Wv, bv, Wo, bo,
     Wc1, bc1, Wc2, bc2, g1, be1, g2, be2, gF, bF, Wp, bp) = args

    xt = jnp.concatenate([jnp.transpose(x_enc, (0, 2, 1)),
                          jnp.transpose(x_mark_enc, (0, 2, 1))], axis=1)
    xt = jnp.pad(xt, ((0, 0), (0, N_PAD - N_TOK), (0, 0))).astype(_BF)

    full = lambda shape: pl.BlockSpec(shape, lambda b: (0,) * len(shape))
    enc_specs = [
        pl.BlockSpec((GB, N_PAD, D_MODEL), lambda b: (b, 0, 0)),
        full((D_MODEL, D_MODEL)), full((1, D_MODEL)),
        full((E_LAYERS, D_MODEL, 3 * D_MODEL)), full((E_LAYERS, 1, 3 * D_MODEL)),
        full((E_LAYERS, D_MODEL, D_MODEL)), full((E_LAYERS, 1, D_MODEL)),
        full((E_LAYERS, D_MODEL, D_FF)), full((E_LAYERS, 1, D_FF)),
        full((E_LAYERS, D_FF, D_MODEL)), full((E_LAYERS, 1, D_MODEL)),
        full((E_LAYERS, 1, D_MODEL)), full((E_LAYERS, 1, D_MODEL)),
        full((E_LAYERS, 1, D_MODEL)), full((E_LAYERS, 1, D_MODEL)),
        full((1, D_MODEL)), full((1, D_MODEL)),
        full((D_MODEL, PRED_LEN)), full((1, PRED_LEN)),
    ]
    pred = pl.pallas_call(
        _enc_kernel,
        grid=(B // GB,),
        in_specs=enc_specs,
        out_specs=pl.BlockSpec((GB, PRED_LEN, N_PAD), lambda b: (b, 0, 0)),
        out_shape=jax.ShapeDtypeStruct((B, PRED_LEN, N_PAD), _F32),
        compiler_params=pltpu.CompilerParams(
            dimension_semantics=("parallel",)),
        interpret=interpret,
    )(
        xt,
        W_emb.astype(_BF), b_emb.reshape(1, D_MODEL),
        jnp.concatenate([Wq, Wk, Wv], axis=2).astype(_BF),
        jnp.concatenate([bq, bk, bv], axis=1).reshape(E_LAYERS, 1, 3 * D_MODEL),
        Wo.astype(_BF), bo.reshape(E_LAYERS, 1, D_MODEL),
        Wc1.astype(_BF), bc1.reshape(E_LAYERS, 1, D_FF),
        Wc2.astype(_BF), bc2.reshape(E_LAYERS, 1, D_MODEL),
        g1.reshape(E_LAYERS, 1, D_MODEL), be1.reshape(E_LAYERS, 1, D_MODEL),
        g2.reshape(E_LAYERS, 1, D_MODEL), be2.reshape(E_LAYERS, 1, D_MODEL),
        gF.reshape(1, D_MODEL), bF.reshape(1, D_MODEL),
        Wp.astype(_BF), bp.reshape(1, PRED_LEN),
    )

    p_flat = pred.reshape(B, FLAT)
    sim = pl.pallas_call(
        _sim_kernel,
        out_shape=jax.ShapeDtypeStruct((B, B), _F32),
        scratch_shapes=[pltpu.VMEM((B, FLAT), _BF)],
        interpret=interpret,
    )(p_flat)

    if interpret:
        col = lax.broadcasted_iota(jnp.int32, (B, B), 1)
        sel = jnp.zeros((B, B), _F32)
        sm = sim
        for _ in range(TOPK):
            m = jnp.max(sm, axis=1, keepdims=True)
            eq = sm == m
            ji = jnp.min(jnp.where(eq, col, B), axis=1, keepdims=True)
            first = col == ji
            sel = jnp.where(first, 1.0, sel)
            sm = jnp.where(first, -jnp.inf, sm)
        A = sel * (1.0 / TOPK)
    else:
        mesh = plsc.VectorSubcoreMesh(core_axis_name="c", subcore_axis_name="s")
        A = pl.kernel(
            _select_sc,
            mesh=mesh,
            out_type=jax.ShapeDtypeStruct((B * B,), _F32),
            scratch_types=[pltpu.VMEM((2 * B,), _F32),
                           pltpu.VMEM((B,), _F32)],
        )(sim.reshape(B * B)).reshape(B, B)

    fused_flat = pl.pallas_call(
        _fuse_kernel,
        out_shape=jax.ShapeDtypeStruct((B, FLAT), _F32),
        interpret=interpret,
    )(p_flat, A)

    fused = fused_flat.reshape(B, PRED_LEN, N_PAD)
    return fused[:, :, :N_VAR]


def kernel(x_enc, x_mark_enc, x_dec, x_mark_dec, W_emb, b_emb, Wq, bq, Wk, bk,
           Wv, bv, Wo, bo, Wc1, bc1, Wc2, bc2, g1, be1, g2, be2, gF, bF,
           Wp, bp):
    return _run((x_enc, x_mark_enc, W_emb, b_emb, Wq, bq, Wk, bk, Wv, bv,
                 Wo, bo, Wc1, bc1, Wc2, bc2, g1, be1, g2, be2, gF, bF,
                 Wp, bp))


# speedup vs baseline: 1.0670x; 1.0670x over previous
import functools

import jax
import jax.numpy as jnp
from jax import lax
from jax.experimental import pallas as pl
from jax.experimental.pallas import tpu as pltpu
from jax.experimental.pallas import tpu_sc as plsc

B = 32
L_SEQ = 512
N_VAR = 321
N_MARK = 4
N_TOK = N_VAR + N_MARK
N_PAD = 328
D_MODEL = 512
D_FF = 2048
N_HEADS = 8
D_HEAD = 64
E_LAYERS = 2
PRED_LEN = 336
TOPK = 10
FLAT = N_PAD * PRED_LEN
CHUNK = 128 * 41
N_CHUNKS = FLAT // CHUNK

GB = 2

_BF = jnp.bfloat16
_F32 = jnp.float32


def _dot(a, b, precision=None):
    return lax.dot_general(a, b, (((1,), (0,)), ((), ())),
                           preferred_element_type=_F32, precision=precision)


def _dot_nt(a, b):
    return lax.dot_general(a, b, (((1,), (1,)), ((), ())),
                           preferred_element_type=_F32)


def _ln(x, g, b, eps=1e-5):
    m = jnp.mean(x, axis=1, keepdims=True)
    v = jnp.mean(x * x, axis=1, keepdims=True) - m * m
    return (x - m) / jnp.sqrt(v + eps) * g + b


def _enc_kernel(xt_ref, Wemb_ref, bemb_ref, Wqkv_ref, bqkv_ref, Wo_ref,
                bo_ref, Wc1_ref, bc1_ref, Wc2_ref, bc2_ref, g1_ref, be1_ref,
                g2_ref, be2_ref, gF_ref, bF_ref, Wp_ref, bp_ref, out_ref):
    x = xt_ref[...].reshape(GB * N_PAD, D_MODEL)
    h = _dot(x, Wemb_ref[...]) + bemb_ref[...]

    col = lax.broadcasted_iota(jnp.int32, (1, N_PAD), 1)
    key_mask = jnp.where(col < N_TOK, 0.0, -1e30).astype(_F32)

    for l in range(E_LAYERS):
        hb = h.astype(_BF)
        qkv = _dot(hb, Wqkv_ref[l]) + bqkv_ref[l]
        qkvb = qkv.astype(_BF)
        qb = qkvb[:, 0:D_MODEL]
        kb = qkvb[:, D_MODEL:2 * D_MODEL]
        vb = qkvb[:, 2 * D_MODEL:3 * D_MODEL]
        rows = []
        for eb in range(GB):
            rs = slice(eb * N_PAD, (eb + 1) * N_PAD)
            heads = []
            for hd in range(N_HEADS):
                sl = slice(hd * D_HEAD, (hd + 1) * D_HEAD)
                s = _dot_nt(qb[rs, sl], kb[rs, sl]) * 0.125
                e = jnp.exp(s + key_mask)
                a = e / jnp.sum(e, axis=1, keepdims=True)
                heads.append(_dot(a.astype(_BF), vb[rs, sl]))
            rows.append(jnp.concatenate(heads, axis=1))
        o = jnp.concatenate(rows, axis=0)
        o = _dot(o.astype(_BF), Wo_ref[l]) + bo_ref[l]
        h = _ln(h + o, g1_ref[l], be1_ref[l])
        y = _dot(h.astype(_BF), Wc1_ref[l]) + bc1_ref[l]
        y = jax.nn.gelu(y)
        y = _dot(y.astype(_BF), Wc2_ref[l]) + bc2_ref[l]
        h = _ln(h + y, g2_ref[l], be2_ref[l])

    h = _ln(h, gF_ref[...], bF_ref[...])
    pt = _dot(h.astype(_BF), Wp_ref[...]) + bp_ref[...]
    row = lax.broadcasted_iota(jnp.int32, (N_PAD, 1), 0)
    pt = pt.reshape(GB, N_PAD, PRED_LEN)
    out_ref[...] = jnp.where(row < N_VAR, pt, 0.0)


def _sim_kernel(p_ref, sim_ref, qnb_ref):
    ss = jnp.zeros((B, 1), _F32)
    for i in range(N_CHUNKS):
        x = p_ref[:, pl.ds(i * CHUNK, CHUNK)]
        ss = ss + jnp.sum(x * x, axis=1, keepdims=True)
    denom = jnp.sqrt(ss) + 1e-8

    for i in range(N_CHUNKS):
        x = p_ref[:, pl.ds(i * CHUNK, CHUNK)]
        qnb_ref[:, pl.ds(i * CHUNK, CHUNK)] = (x / denom).astype(_BF)

    sim = jnp.zeros((B, B), _F32)
    for i in range(N_CHUNKS):
        c = qnb_ref[:, pl.ds(i * CHUNK, CHUNK)]
        sim = sim + _dot_nt(c, c)
    sim_ref[...] = sim


def _select_sc(sim_hbm, a_hbm, row_v, out_v):
    wid = lax.axis_index("s") * 2 + lax.axis_index("c")
    base = wid * B
    pltpu.sync_copy(sim_hbm.at[pl.ds(base, B)], row_v.at[pl.ds(0, B)])
    pltpu.sync_copy(sim_hbm.at[pl.ds(base, B)], row_v.at[pl.ds(B, B)])
    lane = lax.iota(jnp.int32, 16)
    a1 = row_v[pl.ds(0, 16)]
    a2 = row_v[pl.ds(16, 16)]
    r1 = jnp.zeros((16,), jnp.int32)
    r2 = jnp.zeros((16,), jnp.int32)
    one = jnp.ones((16,), jnp.int32)
    zero = jnp.zeros((16,), jnp.int32)
    for sft in range(1, B):
        w1 = row_v[pl.ds(sft, 16)]
        w2 = row_v[pl.ds(16 + sft, 16)]
        beat1 = (w1 > a1) | ((w1 == a1) & (lane >= B - sft))
        beat2 = (w2 > a2) | ((w2 == a2) & (lane >= 16 - sft))
        r1 = r1 + jnp.where(beat1, one, zero)
        r2 = r2 + jnp.where(beat2, one, zero)
    tenth = jnp.full((16,), 1.0 / TOPK, _F32)
    zf = jnp.zeros((16,), _F32)
    out_v[pl.ds(0, 16)] = jnp.where(r1 < TOPK, tenth, zf)
    out_v[pl.ds(16, 16)] = jnp.where(r2 < TOPK, tenth, zf)
    pltpu.sync_copy(out_v, a_hbm.at[pl.ds(base, B)])


def _fuse_kernel(p_ref, a_ref, out_ref):
    A = a_ref[...]
    for i in range(N_CHUNKS):
        x = p_ref[:, pl.ds(i * CHUNK, CHUNK)]
        f = _dot(A, x, precision=lax.Precision.HIGHEST)
        out_ref[:, pl.ds(i * CHUNK, CHUNK)] = 0.5 * x + 0.5 * f


@functools.partial(jax.jit, static_argnames=("interpret",))
def _run(args, interpret=False):
    (x_enc, x_mark_enc, W_emb, b_emb, Wq, bq, Wk, bk, Wv, bv, Wo, bo,
     Wc1, bc1, Wc2, bc2, g1, be1, g2, be2, gF, bF, Wp, bp) = args

    xt = jnp.concatenate([jnp.transpose(x_enc, (0, 2, 1)),
                          jnp.transpose(x_mark_enc, (0, 2, 1))], axis=1)
    xt = jnp.pad(xt, ((0, 0), (0, N_PAD - N_TOK), (0, 0))).astype(_BF)

    full = lambda shape: pl.BlockSpec(shape, lambda b: (0,) * len(shape))
    enc_specs = [
        pl.BlockSpec((GB, N_PAD, D_MODEL), lambda b: (b, 0, 0)),
        full((D_MODEL, D_MODEL)), full((1, D_MODEL)),
        full((E_LAYERS, D_MODEL, 3 * D_MODEL)), full((E_LAYERS, 1, 3 * D_MODEL)),
        full((E_LAYERS, D_MODEL, D_MODEL)), full((E_LAYERS, 1, D_MODEL)),
        full((E_LAYERS, D_MODEL, D_FF)), full((E_LAYERS, 1, D_FF)),
        full((E_LAYERS, D_FF, D_MODEL)), full((E_LAYERS, 1, D_MODEL)),
        full((E_LAYERS, 1, D_MODEL)), full((E_LAYERS, 1, D_MODEL)),
        full((E_LAYERS, 1, D_MODEL)), full((E_LAYERS, 1, D_MODEL)),
        full((1, D_MODEL)), full((1, D_MODEL)),
        full((D_MODEL, PRED_LEN)), full((1, PRED_LEN)),
    ]
    pred = pl.pallas_call(
        _enc_kernel,
        grid=(B // GB,),
        in_specs=enc_specs,
        out_specs=pl.BlockSpec((GB, N_PAD, PRED_LEN), lambda b: (b, 0, 0)),
        out_shape=jax.ShapeDtypeStruct((B, N_PAD, PRED_LEN), _F32),
        compiler_params=pltpu.CompilerParams(
            dimension_semantics=("arbitrary",)),
        interpret=interpret,
    )(
        xt,
        W_emb.astype(_BF), b_emb.reshape(1, D_MODEL),
        jnp.concatenate([Wq, Wk, Wv], axis=2).astype(_BF),
        jnp.concatenate([bq, bk, bv], axis=1).reshape(E_LAYERS, 1, 3 * D_MODEL),
        Wo.astype(_BF), bo.reshape(E_LAYERS, 1, D_MODEL),
        Wc1.astype(_BF), bc1.reshape(E_LAYERS, 1, D_FF),
        Wc2.astype(_BF), bc2.reshape(E_LAYERS, 1, D_MODEL),
        g1.reshape(E_LAYERS, 1, D_MODEL), be1.reshape(E_LAYERS, 1, D_MODEL),
        g2.reshape(E_LAYERS, 1, D_MODEL), be2.reshape(E_LAYERS, 1, D_MODEL),
        gF.reshape(1, D_MODEL), bF.reshape(1, D_MODEL),
        Wp.astype(_BF), bp.reshape(1, PRED_LEN),
    )

    p_flat = pred.reshape(B, FLAT)
    sim = pl.pallas_call(
        _sim_kernel,
        out_shape=jax.ShapeDtypeStruct((B, B), _F32),
        scratch_shapes=[pltpu.VMEM((B, FLAT), _BF)],
        interpret=interpret,
    )(p_flat)

    if interpret:
        col = lax.broadcasted_iota(jnp.int32, (B, B), 1)
        sel = jnp.zeros((B, B), _F32)
        sm = sim
        for _ in range(TOPK):
            m = jnp.max(sm, axis=1, keepdims=True)
            eq = sm == m
            ji = jnp.min(jnp.where(eq, col, B), axis=1, keepdims=True)
            first = col == ji
            sel = jnp.where(first, 1.0, sel)
            sm = jnp.where(first, -jnp.inf, sm)
        A = sel * (1.0 / TOPK)
    else:
        mesh = plsc.VectorSubcoreMesh(core_axis_name="c", subcore_axis_name="s")
        A = pl.kernel(
            _select_sc,
            mesh=mesh,
            out_type=jax.ShapeDtypeStruct((B * B,), _F32),
            scratch_types=[pltpu.VMEM((2 * B,), _F32),
                           pltpu.VMEM((B,), _F32)],
        )(sim.reshape(B * B)).reshape(B, B)

    fused_flat = pl.pallas_call(
        _fuse_kernel,
        out_shape=jax.ShapeDtypeStruct((B, FLAT), _F32),
        interpret=interpret,
    )(p_flat, A)

    fused = fused_flat.reshape(B, N_PAD, PRED_LEN)
    return jnp.transpose(fused, (0, 2, 1))[:, :, :N_VAR]


def kernel(x_enc, x_mark_enc, x_dec, x_mark_dec, W_emb, b_emb, Wq, bq, Wk, bk,
           Wv, bv, Wo, bo, Wc1, bc1, Wc2, bc2, g1, be1, g2, be2, gF, bF,
           Wp, bp):
    return _run((x_enc, x_mark_enc, W_emb, b_emb, Wq, bq, Wk, bk, Wv, bv,
                 Wo, bo, Wc1, bc1, Wc2, bc2, g1, be1, g2, be2, gF, bF,
                 Wp, bp))
